# baseline (speedup 1.0000x reference)
import functools

import jax
import jax.numpy as jnp
from jax import lax
from jax.experimental import pallas as pl
from jax.experimental.pallas import tpu as pltpu
from jax.experimental.pallas import tpu_sc as plsc

B = 16384
VOCAB = 1000000
ED = 32
HIST = 50
NLIN = 13
HID = 256

NC, NS, L = 2, 16, 16
NW = NC * NS
BPW = B // NW
CB = 16
NCH = BPW // CB
PACK = 4
VP = VOCAB // PACK


def _sc_gather_pool(user, item, hist_flat, eu_p, ei_p, eh_p):
    mesh = plsc.VectorSubcoreMesh(core_axis_name="c", subcore_axis_name="s")

    @functools.partial(
        pl.kernel,
        out_type=jax.ShapeDtypeStruct((B, 4 * ED), jnp.float32),
        mesh=mesh,
        scratch_types=[
            pltpu.VMEM((CB,), jnp.int32),
            pltpu.VMEM((CB,), jnp.int32),
            pltpu.VMEM((CB * HIST,), jnp.int32),
            pltpu.VMEM((CB,), jnp.int32),
            pltpu.VMEM((CB,), jnp.int32),
            pltpu.VMEM((CB * HIST,), jnp.int32),
            pltpu.VMEM((CB * HIST,), jnp.int32),
            pltpu.VMEM((CB, 128), jnp.float32),
            pltpu.VMEM((CB, 128), jnp.float32),
            pltpu.VMEM((CB * HIST, 128), jnp.float32),
            pltpu.VMEM((CB, 128), jnp.float32),
            pltpu.SemaphoreType.DMA,
            pltpu.SemaphoreType.DMA,
            pltpu.SemaphoreType.DMA,
        ],
        compiler_params=pltpu.CompilerParams(needs_layout_passes=False),
    )
    def k(user_h, item_h, hist_h, eu_h, ei_h, eh_h, out_h,
          uidx_v, iidx_v, hidx_v, u4_v, i4_v, h4_v, hsub_v,
          rows_u, rows_i, rows_h, stage_v, semu, semi, semh):
        wid = lax.axis_index("s") * NC + lax.axis_index("c")
        base = wid * BPW
        iota = lax.iota(jnp.int32, L)
        riota = iota * HIST

        def chunk_body(ch, carry):
            row0 = base + ch * CB
            pltpu.sync_copy(user_h.at[pl.ds(row0, CB)], uidx_v)
            pltpu.sync_copy(item_h.at[pl.ds(row0, CB)], iidx_v)
            pltpu.sync_copy(hist_h.at[pl.ds(row0 * HIST, CB * HIST)], hidx_v)
            uv = uidx_v[...]
            u4_v[...] = jnp.right_shift(uv, 2)
            usub = jnp.left_shift(jnp.bitwise_and(uv, 3), 5)
            iv = iidx_v[...]
            i4_v[...] = jnp.right_shift(iv, 2)
            isub = jnp.left_shift(jnp.bitwise_and(iv, 3), 5)
            for g in range(HIST):
                hv = hidx_v[pl.ds(g * L, L)]
                h4_v[pl.ds(g * L, L)] = jnp.right_shift(hv, 2)
                hsub_v[pl.ds(g * L, L)] = jnp.left_shift(
                    jnp.bitwise_and(hv, 3), 5)
            cu = pltpu.async_copy(eu_h.at[u4_v], rows_u, semu)
            ci = pltpu.async_copy(ei_h.at[i4_v], rows_i, semi)
            chh = pltpu.async_copy(eh_h.at[h4_v], rows_h, semh)
            cu.wait()
            for c in range(ED):
                v = plsc.load_gather(rows_u, [iota, usub + c])
                plsc.store_scatter(
                    stage_v, [iota, jnp.full((L,), c, jnp.int32)], v)
            ci.wait()
            for c in range(ED):
                v = plsc.load_gather(rows_i, [iota, isub + c])
                plsc.store_scatter(
                    stage_v, [iota, jnp.full((L,), ED + c, jnp.int32)], v)
            chh.wait()

            def jbody(j, accs):
                row_j = riota + j
                sub_j = plsc.load_gather(hsub_v, [row_j])
                return tuple(
                    accs[c] + plsc.load_gather(rows_h, [row_j, sub_j + c])
                    for c in range(ED))

            accs = lax.fori_loop(
                0, HIST, jbody,
                tuple(jnp.zeros((L,), jnp.float32) for _ in range(ED)))
            for c in range(ED):
                plsc.store_scatter(
                    stage_v, [iota, jnp.full((L,), 2 * ED + c, jnp.int32)],
                    accs[c])
            pltpu.sync_copy(stage_v, out_h.at[pl.ds(row0, CB)])
            return carry

        lax.fori_loop(0, NCH, chunk_body, 0)

    return k(user, item, hist_flat, eu_p, ei_p, eh_p)


BT = 2048


def _tc_mlp(comb, dense, wd_t, bd, w1c, w1d, b1, w2_t, b2):
    grid = (B // BT,)

    def body(comb_ref, d_ref, wd_ref, bd_ref, w1c_ref, w1d_ref, b1_ref,
             w2_ref, b2_ref, o_ref):
        embd = jnp.dot(d_ref[...], wd_ref[...],
                       preferred_element_type=jnp.float32) + bd_ref[...]
        h1 = (jnp.dot(comb_ref[...], w1c_ref[...],
                      preferred_element_type=jnp.float32)
              + jnp.dot(embd, w1d_ref[...],
                        preferred_element_type=jnp.float32)
              + b1_ref[...])
        h1 = jnp.maximum(h1, 0.0)
        o_ref[...] = jnp.dot(h1, w2_ref[...],
                             preferred_element_type=jnp.float32) + b2_ref[...]

    batch_spec = lambda d: pl.BlockSpec((BT, d), lambda i: (i, 0))
    full = lambda a: pl.BlockSpec(a.shape, lambda i: (0,) * a.ndim)

    return pl.pallas_call(
        body,
        grid=grid,
        in_specs=[
            batch_spec(4 * ED), batch_spec(NLIN),
            full(wd_t), full(bd), full(w1c), full(w1d),
            full(b1), full(w2_t), full(b2),
        ],
        out_specs=pl.BlockSpec((BT, 1), lambda i: (i, 0)),
        out_shape=jax.ShapeDtypeStruct((B, 1), jnp.float32),
    )(comb, dense, wd_t, bd, w1c, w1d, b1, w2_t, b2)


def kernel(user, item, hist, dense, E_user, E_item, E_hist,
           W_dense, b_dense, W1, b1, W2, b2):
    comb = _sc_gather_pool(
        user.astype(jnp.int32), item.astype(jnp.int32),
        hist.reshape(-1).astype(jnp.int32),
        E_user.reshape(VP, PACK * ED), E_item.reshape(VP, PACK * ED),
        E_hist.reshape(VP, PACK * ED))
    w1_t = W1.T
    w1c = jnp.concatenate(
        [w1_t[:3 * ED], jnp.zeros((ED, HID), jnp.float32)], axis=0)
    return _tc_mlp(
        comb, dense, W_dense.T, b_dense.reshape(1, ED),
        w1c, w1_t[3 * ED:4 * ED],
        b1.reshape(1, HID), W2.T, b2.reshape(1, 1))


# speedup vs baseline: 1.4695x; 1.4695x over previous
import functools

import jax
import jax.numpy as jnp
from jax import lax
from jax.experimental import pallas as pl
from jax.experimental.pallas import tpu as pltpu
from jax.experimental.pallas import tpu_sc as plsc

B = 16384
VOCAB = 1000000
ED = 32
HIST = 50
NLIN = 13
HID = 256

NC, NS, L = 2, 16, 16
NW = NC * NS
BPW = B // NW
CB = 16
NCH = BPW // CB


def _sc_gather_pool(user, item, hist_flat, E_user, E_item, E_hist):
    mesh = plsc.VectorSubcoreMesh(core_axis_name="c", subcore_axis_name="s")

    @functools.partial(
        pl.kernel,
        out_type=[
            jax.ShapeDtypeStruct((B, ED), jnp.float32),
            jax.ShapeDtypeStruct((B, ED), jnp.float32),
            jax.ShapeDtypeStruct((B, ED), jnp.float32),
        ],
        mesh=mesh,
        scratch_types=[
            pltpu.VMEM((BPW,), jnp.int32),
            pltpu.VMEM((BPW,), jnp.int32),
            pltpu.VMEM((BPW, ED), jnp.float32),
            pltpu.VMEM((BPW, ED), jnp.float32),
            pltpu.VMEM((CB * HIST,), jnp.int32),
            pltpu.VMEM((CB * HIST,), jnp.int32),
            pltpu.VMEM((CB * HIST, ED), jnp.float32),
            pltpu.VMEM((CB * HIST, ED), jnp.float32),
            pltpu.VMEM((CB, ED), jnp.float32),
            pltpu.SemaphoreType.DMA,
            pltpu.SemaphoreType.DMA,
            pltpu.SemaphoreType.DMA,
            pltpu.SemaphoreType.DMA,
        ],
        compiler_params=pltpu.CompilerParams(use_tc_tiling_on_sc=False),
    )
    def k(user_h, item_h, hist_h, eu_h, ei_h, eh_h, out_u, out_i, out_h,
          uidx_v, iidx_v, rows_u, rows_i, hidx_a, hidx_b, hrows_a, hrows_b,
          acc_v, sem_u, sem_i, sem_a, sem_b):
        wid = lax.axis_index("s") * NC + lax.axis_index("c")
        base = wid * BPW

        pltpu.sync_copy(user_h.at[pl.ds(base, BPW)], uidx_v)
        cu = pltpu.async_copy(eu_h.at[uidx_v], rows_u, sem_u)
        pltpu.sync_copy(item_h.at[pl.ds(base, BPW)], iidx_v)
        ci = pltpu.async_copy(ei_h.at[iidx_v], rows_i, sem_i)

        def start_hist(c, hidx_v, hrows_v, sem):
            cc = jnp.minimum(c, NCH - 1)
            row0 = (base + cc * CB) * HIST
            pltpu.sync_copy(hist_h.at[pl.ds(row0, CB * HIST)], hidx_v)
            pltpu.async_copy(eh_h.at[hidx_v], hrows_v, sem)

        def wait_hist(hidx_v, hrows_v, sem):
            pltpu.make_async_copy(eh_h.at[hidx_v], hrows_v, sem).wait()

        def pool_chunk(c, hrows_v):
            def row_body(b, carry):
                a0 = jnp.zeros((L,), jnp.float32)
                a1 = jnp.zeros((L,), jnp.float32)
                for j in range(HIST):
                    a0 = a0 + hrows_v[b * HIST + j, 0:L]
                    a1 = a1 + hrows_v[b * HIST + j, L:2 * L]
                acc_v[b, 0:L] = a0
                acc_v[b, L:2 * L] = a1
                return carry

            lax.fori_loop(0, CB, row_body, 0)
            pltpu.sync_copy(acc_v, out_h.at[pl.ds(base + c * CB, CB)])

        start_hist(0, hidx_a, hrows_a, sem_a)
        start_hist(1, hidx_b, hrows_b, sem_b)

        def pair_body(i, carry):
            a = 2 * i
            wait_hist(hidx_a, hrows_a, sem_a)
            pool_chunk(a, hrows_a)
            start_hist(a + 2, hidx_a, hrows_a, sem_a)
            wait_hist(hidx_b, hrows_b, sem_b)
            pool_chunk(a + 1, hrows_b)
            start_hist(a + 3, hidx_b, hrows_b, sem_b)
            return carry

        lax.fori_loop(0, NCH // 2, pair_body, 0)
        wait_hist(hidx_a, hrows_a, sem_a)
        wait_hist(hidx_b, hrows_b, sem_b)
        cu.wait()
        pltpu.sync_copy(rows_u, out_u.at[pl.ds(base, BPW)])
        ci.wait()
        pltpu.sync_copy(rows_i, out_i.at[pl.ds(base, BPW)])

    return k(user, item, hist_flat, E_user, E_item, E_hist)


BT = 2048


def _tc_mlp(eu, ei, eh, dense, wd_t, bd, w1u, w1i, w1h, w1d, b1, w2_t, b2):
    grid = (B // BT,)

    def body(eu_ref, ei_ref, eh_ref, d_ref, wd_ref, bd_ref,
             w1u_ref, w1i_ref, w1h_ref, w1d_ref, b1_ref, w2_ref, b2_ref,
             o_ref):
        embd = jnp.dot(d_ref[...], wd_ref[...],
                       preferred_element_type=jnp.float32) + bd_ref[...]
        h1 = (jnp.dot(eu_ref[...], w1u_ref[...],
                      preferred_element_type=jnp.float32)
              + jnp.dot(ei_ref[...], w1i_ref[...],
                        preferred_element_type=jnp.float32)
              + jnp.dot(eh_ref[...], w1h_ref[...],
                        preferred_element_type=jnp.float32)
              + jnp.dot(embd, w1d_ref[...],
                        preferred_element_type=jnp.float32)
              + b1_ref[...])
        h1 = jnp.maximum(h1, 0.0)
        o_ref[...] = jnp.dot(h1, w2_ref[...],
                             preferred_element_type=jnp.float32) + b2_ref[...]

    batch_spec = lambda d: pl.BlockSpec((BT, d), lambda i: (i, 0))
    full = lambda a: pl.BlockSpec(a.shape, lambda i: (0,) * a.ndim)

    return pl.pallas_call(
        body,
        grid=grid,
        in_specs=[
            batch_spec(ED), batch_spec(ED), batch_spec(ED), batch_spec(NLIN),
            full(wd_t), full(bd), full(w1u), full(w1i), full(w1h), full(w1d),
            full(b1), full(w2_t), full(b2),
        ],
        out_specs=pl.BlockSpec((BT, 1), lambda i: (i, 0)),
        out_shape=jax.ShapeDtypeStruct((B, 1), jnp.float32),
    )(eu, ei, eh, dense, wd_t, bd, w1u, w1i, w1h, w1d, b1, w2_t, b2)


def kernel(user, item, hist, dense, E_user, E_item, E_hist,
           W_dense, b_dense, W1, b1, W2, b2):
    eu, ei, eh = _sc_gather_pool(
        user.astype(jnp.int32), item.astype(jnp.int32),
        hist.reshape(-1).astype(jnp.int32), E_user, E_item, E_hist)
    w1_t = W1.T
    return _tc_mlp(
        eu, ei, eh, dense,
        W_dense.T, b_dense.reshape(1, ED),
        w1_t[0 * ED:1 * ED], w1_t[1 * ED:2 * ED],
        w1_t[2 * ED:3 * ED], w1_t[3 * ED:4 * ED],
        b1.reshape(1, HID), W2.T, b2.reshape(1, 1))


# speedup vs baseline: 1.5165x; 1.0320x over previous
import functools

import jax
import jax.numpy as jnp
from jax import lax
from jax.experimental import pallas as pl
from jax.experimental.pallas import tpu as pltpu
from jax.experimental.pallas import tpu_sc as plsc

B = 16384
VOCAB = 1000000
ED = 32
HIST = 50
NLIN = 13
HID = 256

NC, NS, L = 2, 16, 16
NW = NC * NS
BPW = B // NW
CB = 16
NCH = BPW // CB


def _sc_gather(idx, table):
    mesh = plsc.VectorSubcoreMesh(core_axis_name="c", subcore_axis_name="s")

    @functools.partial(
        pl.kernel,
        out_type=jax.ShapeDtypeStruct((B, ED), jnp.float32),
        mesh=mesh,
        scratch_types=[
            pltpu.VMEM((BPW,), jnp.int32),
            pltpu.VMEM((BPW, ED), jnp.float32),
            pltpu.SemaphoreType.DMA,
        ],
        compiler_params=pltpu.CompilerParams(use_tc_tiling_on_sc=False),
    )
    def k(idx_h, tab_h, out_h, idx_v, rows_v, sem):
        wid = lax.axis_index("s") * NC + lax.axis_index("c")
        base = wid * BPW
        pltpu.sync_copy(idx_h.at[pl.ds(base, BPW)], idx_v)
        pltpu.async_copy(tab_h.at[idx_v], rows_v, sem).wait()
        pltpu.sync_copy(rows_v, out_h.at[pl.ds(base, BPW)])

    return k(idx, table)


def _sc_hist_pool(hist_flat, E_hist):
    mesh = plsc.VectorSubcoreMesh(core_axis_name="c", subcore_axis_name="s")

    @functools.partial(
        pl.kernel,
        out_type=jax.ShapeDtypeStruct((B, ED), jnp.float32),
        mesh=mesh,
        scratch_types=[
            pltpu.VMEM((CB * HIST,), jnp.int32),
            pltpu.VMEM((CB * HIST,), jnp.int32),
            pltpu.VMEM((CB * HIST, ED), jnp.float32),
            pltpu.VMEM((CB * HIST, ED), jnp.float32),
            pltpu.VMEM((CB, ED), jnp.float32),
            pltpu.SemaphoreType.DMA,
            pltpu.SemaphoreType.DMA,
        ],
        compiler_params=pltpu.CompilerParams(use_tc_tiling_on_sc=False),
    )
    def k(hist_h, eh_h, out_h,
          hidx_a, hidx_b, hrows_a, hrows_b, acc_v, sem_a, sem_b):
        wid = lax.axis_index("s") * NC + lax.axis_index("c")
        base = wid * BPW

        def start_hist(c, hidx_v, hrows_v, sem):
            cc = jnp.minimum(c, NCH - 1)
            row0 = (base + cc * CB) * HIST
            pltpu.sync_copy(hist_h.at[pl.ds(row0, CB * HIST)], hidx_v)
            pltpu.async_copy(eh_h.at[hidx_v], hrows_v, sem)

        def wait_hist(hidx_v, hrows_v, sem):
            pltpu.make_async_copy(eh_h.at[hidx_v], hrows_v, sem).wait()

        def pool_chunk(c, hrows_v):
            def row_body(b, carry):
                a0 = jnp.zeros((L,), jnp.float32)
                a1 = jnp.zeros((L,), jnp.float32)
                for j in range(HIST):
                    a0 = a0 + hrows_v[b * HIST + j, 0:L]
                    a1 = a1 + hrows_v[b * HIST + j, L:2 * L]
                acc_v[b, 0:L] = a0
                acc_v[b, L:2 * L] = a1
                return carry

            lax.fori_loop(0, CB, row_body, 0)
            pltpu.sync_copy(acc_v, out_h.at[pl.ds(base + c * CB, CB)])

        start_hist(0, hidx_a, hrows_a, sem_a)
        start_hist(1, hidx_b, hrows_b, sem_b)

        def pair_body(i, carry):
            a = 2 * i
            wait_hist(hidx_a, hrows_a, sem_a)
            pool_chunk(a, hrows_a)
            start_hist(a + 2, hidx_a, hrows_a, sem_a)
            wait_hist(hidx_b, hrows_b, sem_b)
            pool_chunk(a + 1, hrows_b)
            start_hist(a + 3, hidx_b, hrows_b, sem_b)
            return carry

        lax.fori_loop(0, NCH // 2, pair_body, 0)
        wait_hist(hidx_a, hrows_a, sem_a)
        wait_hist(hidx_b, hrows_b, sem_b)

    return k(hist_flat, E_hist)


BT = 2048


def _tc_mlp(eu, ei, eh, dense, wd_t, bd, w1u, w1i, w1h, w1d, b1, w2_t, b2):
    grid = (B // BT,)

    def body(eu_ref, ei_ref, eh_ref, d_ref, wd_ref, bd_ref,
             w1u_ref, w1i_ref, w1h_ref, w1d_ref, b1_ref, w2_ref, b2_ref,
             o_ref):
        embd = jnp.dot(d_ref[...], wd_ref[...],
                       preferred_element_type=jnp.float32) + bd_ref[...]
        h1 = (jnp.dot(eu_ref[...], w1u_ref[...],
                      preferred_element_type=jnp.float32)
              + jnp.dot(ei_ref[...], w1i_ref[...],
                        preferred_element_type=jnp.float32)
              + jnp.dot(eh_ref[...], w1h_ref[...],
                        preferred_element_type=jnp.float32)
              + jnp.dot(embd, w1d_ref[...],
                        preferred_element_type=jnp.float32)
              + b1_ref[...])
        h1 = jnp.maximum(h1, 0.0)
        o_ref[...] = jnp.dot(h1, w2_ref[...],
                             preferred_element_type=jnp.float32) + b2_ref[...]

    batch_spec = lambda d: pl.BlockSpec((BT, d), lambda i: (i, 0))
    full = lambda a: pl.BlockSpec(a.shape, lambda i: (0,) * a.ndim)

    return pl.pallas_call(
        body,
        grid=grid,
        in_specs=[
            batch_spec(ED), batch_spec(ED), batch_spec(ED), batch_spec(NLIN),
            full(wd_t), full(bd), full(w1u), full(w1i), full(w1h), full(w1d),
            full(b1), full(w2_t), full(b2),
        ],
        out_specs=pl.BlockSpec((BT, 1), lambda i: (i, 0)),
        out_shape=jax.ShapeDtypeStruct((B, 1), jnp.float32),
    )(eu, ei, eh, dense, wd_t, bd, w1u, w1i, w1h, w1d, b1, w2_t, b2)


def kernel(user, item, hist, dense, E_user, E_item, E_hist,
           W_dense, b_dense, W1, b1, W2, b2):
    eu = _sc_gather(user.astype(jnp.int32), E_user)
    ei = _sc_gather(item.astype(jnp.int32), E_item)
    eh = _sc_hist_pool(hist.reshape(-1).astype(jnp.int32), E_hist)
    w1_t = W1.T
    return _tc_mlp(
        eu, ei, eh, dense,
        W_dense.T, b_dense.reshape(1, ED),
        w1_t[0 * ED:1 * ED], w1_t[1 * ED:2 * ED],
        w1_t[2 * ED:3 * ED], w1_t[3 * ED:4 * ED],
        b1.reshape(1, HID), W2.T, b2.reshape(1, 1))
